# Initial kernel scaffold; baseline (speedup 1.0000x reference)
import jax
import jax.numpy as jnp
from jax.experimental import pallas as pl


def kernel(x, edge_index, batch, W_in, b_in, Wg, bg, Wc1, bc1, Wc2, bc2, Wc3, bc3):
    raise NotImplementedError("write your pallas kernel here")


# speedup vs baseline: 14.8309x; 14.8309x over previous
import functools

import jax
import jax.numpy as jnp
from jax import lax
from jax.experimental import pallas as pl
from jax.experimental.pallas import tpu as pltpu
from jax.experimental.pallas import tpu_sc as plsc

_N, _E, _H, _G, _C, _L = 10000, 320000, 128, 64, 5, 4
_NC, _NS = 2, 16
_NW = _NC * _NS
_K = 128
_CH = -(-_E // (_NW * _K))
_EPT = _CH * _K
_EP = _EPT * _NW
_NPAD = 10240
_RPS = _NPAD // _NS
_DW = 128

@functools.cache
def _mesh():
    return plsc.VectorSubcoreMesh(core_axis_name="c", subcore_axis_name="s")


def _sc_degree(dst_chunks, ones_rows, zeros_rows):
    @functools.partial(
        pl.kernel,
        out_type=jax.ShapeDtypeStruct((_NC, _NPAD, _DW), jnp.float32),
        mesh=_mesh(),
        scratch_types=[
            pltpu.VMEM((_CH, _K), jnp.int32),
            pltpu.VMEM((_K, _DW), jnp.float32),
            pltpu.VMEM_SHARED((_NPAD, _DW), jnp.float32),
        ],
    )
    def k(dst_hbm, ones_hbm, zer_hbm, out_hbm, idx_v, ones_v, acc):
        c = lax.axis_index("c")
        s = lax.axis_index("s")
        wid = c * _NS + s
        pltpu.sync_copy(dst_hbm.at[wid], idx_v)
        pltpu.sync_copy(ones_hbm, ones_v)
        pltpu.sync_copy(zer_hbm, acc.at[pl.ds(s * _RPS, _RPS)])
        plsc.subcore_barrier()

        @pl.loop(0, _CH)
        def _(ch):
            pltpu.sync_copy(ones_v, acc.at[idx_v.at[ch]], add=True)

        plsc.subcore_barrier()
        pltpu.sync_copy(acc.at[pl.ds(s * _RPS, _RPS)],
                        out_hbm.at[c, pl.ds(s * _RPS, _RPS)])

    return k(dst_chunks, ones_rows, zeros_rows)


def _sc_msgpass(y, src_chunks, dst_chunks, zeros_rows):
    @functools.partial(
        pl.kernel,
        out_type=jax.ShapeDtypeStruct((_NC, _NPAD, _H), jnp.float32),
        mesh=_mesh(),
        scratch_types=[
            pltpu.VMEM((_CH, _K), jnp.int32),
            pltpu.VMEM((_CH, _K), jnp.int32),
            pltpu.VMEM((_K, _H), jnp.float32),
            pltpu.VMEM_SHARED((_NPAD, _H), jnp.float32),
            pltpu.SemaphoreType.DMA,
        ],
    )
    def k(y_hbm, src_hbm, dst_hbm, zer_hbm, out_hbm,
          src_v, dst_v, rows0, acc, sem):
        c = lax.axis_index("c")
        s = lax.axis_index("s")
        wid = c * _NS + s
        pltpu.sync_copy(src_hbm.at[wid], src_v)
        pltpu.sync_copy(dst_hbm.at[wid], dst_v)
        pltpu.sync_copy(zer_hbm, acc.at[pl.ds(s * _RPS, _RPS)])
        plsc.subcore_barrier()

        @pl.loop(0, _CH)
        def _(ch):
            pltpu.async_copy(y_hbm.at[src_v.at[ch]], rows0, sem).wait()
            pltpu.sync_copy(rows0, acc.at[dst_v.at[ch]], add=True)

        plsc.subcore_barrier()
        pltpu.sync_copy(acc.at[pl.ds(s * _RPS, _RPS)],
                        out_hbm.at[c, pl.ds(s * _RPS, _RPS)])

    return k(y, src_chunks, dst_chunks, zeros_rows)


def _tc_prologue(xp, W_in, b_in, Wg0, degp):
    br = 1280
    nb = _NPAD // br

    def body(x_ref, w_ref, b_ref, wg_ref, degp_ref, dinv_ref, h0_ref, y0_ref):
        deg = 1.0 + degp_ref[0, :, 0:1] + degp_ref[1, :, 0:1]
        dinv = lax.rsqrt(deg)
        dinv_ref[...] = dinv
        h0 = jnp.dot(x_ref[...], w_ref[...],
                     preferred_element_type=jnp.float32) + b_ref[...]
        h0_ref[...] = h0
        y0_ref[...] = jnp.dot(h0, wg_ref[...],
                              preferred_element_type=jnp.float32) * dinv

    return pl.pallas_call(
        body,
        grid=(nb,),
        in_specs=[
            pl.BlockSpec((br, _H), lambda i: (i, 0)),
            pl.BlockSpec((_H, _H), lambda i: (0, 0)),
            pl.BlockSpec((1, _H), lambda i: (0, 0)),
            pl.BlockSpec((_H, _H), lambda i: (0, 0)),
            pl.BlockSpec((_NC, br, _DW), lambda i: (0, i, 0)),
        ],
        out_specs=[
            pl.BlockSpec((br, 1), lambda i: (i, 0)),
            pl.BlockSpec((br, _H), lambda i: (i, 0)),
            pl.BlockSpec((br, _H), lambda i: (i, 0)),
        ],
        out_shape=[
            jax.ShapeDtypeStruct((_NPAD, 1), jnp.float32),
            jax.ShapeDtypeStruct((_NPAD, _H), jnp.float32),
            jax.ShapeDtypeStruct((_NPAD, _H), jnp.float32),
        ],
    )(xp, W_in, b_in, Wg0, degp)


def _tc_combine(p, y, h, dinv, bg_i, Wg_next, project):
    br = 1280
    nb = _NPAD // br

    def body(p_ref, y_ref, h_ref, dinv_ref, bg_ref, wg_ref, *outs):
        dinv = dinv_ref[...]
        agg = p_ref[0] + p_ref[1] + y_ref[...]
        hn = jnp.maximum(dinv * agg + bg_ref[...] + h_ref[...], 0.0)
        outs[0][...] = hn
        if project:
            outs[1][...] = jnp.dot(hn, wg_ref[...],
                                   preferred_element_type=jnp.float32) * dinv

    out_specs = [pl.BlockSpec((br, _H), lambda i: (i, 0))]
    out_shape = [jax.ShapeDtypeStruct((_NPAD, _H), jnp.float32)]
    if project:
        out_specs = out_specs * 2
        out_shape = out_shape * 2
    return pl.pallas_call(
        body,
        grid=(nb,),
        in_specs=[
            pl.BlockSpec((_NC, br, _H), lambda i: (0, i, 0)),
            pl.BlockSpec((br, _H), lambda i: (i, 0)),
            pl.BlockSpec((br, _H), lambda i: (i, 0)),
            pl.BlockSpec((br, 1), lambda i: (i, 0)),
            pl.BlockSpec((1, _H), lambda i: (0, 0)),
            pl.BlockSpec((_H, _H), lambda i: (0, 0)),
        ],
        out_specs=out_specs,
        out_shape=out_shape,
    )(p, y, h, dinv, bg_i, Wg_next)


def _tc_pool_mlp(h, batch_p, Wc1, bc1, Wc2, bc2, Wc3p, bc3p):
    br = 1280
    nb = _NPAD // br
    neg_inf = float("-inf")

    def body(h_ref, b_ref, wc1_ref, bc1_ref, wc2_ref, bc2_ref, wc3_ref,
             bc3_ref, out_ref, sums, cnts, mx):
        i = pl.program_id(0)

        @pl.when(i == 0)
        def _():
            sums[...] = jnp.zeros((_G, _H), jnp.float32)
            cnts[...] = jnp.zeros((_G, _H), jnp.float32)
            mx[...] = jnp.full((_G, _H), neg_inf, jnp.float32)

        hb = h_ref[...]
        bb = b_ref[...]
        gids = lax.broadcasted_iota(jnp.int32, (1, _G), 1)
        m = (bb == gids).astype(jnp.float32)
        dn = (((0,), (0,)), ((), ()))
        sums[...] += lax.dot_general(m, hb, dn,
                                     preferred_element_type=jnp.float32)
        cnts[...] += lax.dot_general(m, jnp.ones_like(hb), dn,
                                     preferred_element_type=jnp.float32)
        seg = [jnp.max(jnp.where(bb == g, hb, neg_inf), axis=0)[None, :]
               for g in range(_G)]
        mx[...] = jnp.maximum(mx[...], jnp.concatenate(seg, axis=0))

        @pl.when(i == nb - 1)
        def _():
            mean = sums[...] / jnp.maximum(cnts[...], 1.0)
            pooled = jnp.concatenate([mean, mx[...]], axis=1)
            z = jnp.dot(pooled, wc1_ref[...],
                        preferred_element_type=jnp.float32) + bc1_ref[...]
            z = jnp.maximum(z, 0.0)
            z = jnp.dot(z, wc2_ref[...],
                        preferred_element_type=jnp.float32) + bc2_ref[...]
            z = jnp.maximum(z, 0.0)
            out_ref[...] = jnp.dot(z, wc3_ref[...],
                                   preferred_element_type=jnp.float32) + bc3_ref[...]

    return pl.pallas_call(
        body,
        grid=(nb,),
        in_specs=[
            pl.BlockSpec((br, _H), lambda i: (i, 0)),
            pl.BlockSpec((br, 1), lambda i: (i, 0)),
            pl.BlockSpec((2 * _H, _H), lambda i: (0, 0)),
            pl.BlockSpec((1, _H), lambda i: (0, 0)),
            pl.BlockSpec((_H, _G), lambda i: (0, 0)),
            pl.BlockSpec((1, _G), lambda i: (0, 0)),
            pl.BlockSpec((_G, _H), lambda i: (0, 0)),
            pl.BlockSpec((1, _H), lambda i: (0, 0)),
        ],
        out_specs=pl.BlockSpec((_G, _H), lambda i: (0, 0)),
        out_shape=jax.ShapeDtypeStruct((_G, _H), jnp.float32),
        scratch_shapes=[
            pltpu.VMEM((_G, _H), jnp.float32),
            pltpu.VMEM((_G, _H), jnp.float32),
            pltpu.VMEM((_G, _H), jnp.float32),
        ],
    )(h, batch_p, Wc1, bc1, Wc2, bc2, Wc3p, bc3p)


def kernel(x, edge_index, batch, W_in, b_in, Wg, bg, Wc1, bc1, Wc2, bc2,
           Wc3, bc3):
    src = edge_index[0]
    dst = edge_index[1]
    npad_rows = _NPAD - _N
    pad_e = _EP - _E
    pad_src = (jnp.arange(pad_e, dtype=jnp.int32) % 1024)
    pad_dst = _N + (jnp.arange(pad_e, dtype=jnp.int32) % npad_rows)
    src_chunks = jnp.concatenate([src, pad_src]).reshape(_NW, _CH, _K)
    dst_chunks = jnp.concatenate([dst, pad_dst]).reshape(_NW, _CH, _K)

    ones_rows = jnp.ones((_K, _DW), jnp.float32)
    zeros_rows = jnp.zeros((_RPS, _H), jnp.float32)

    xp = jnp.pad(x, ((0, npad_rows), (0, 0)))
    batch_p = jnp.pad(batch, (0, npad_rows), constant_values=_G)
    batch_p = batch_p.reshape(_NPAD, 1)

    b_in2 = b_in.reshape(1, _H)
    bg2 = bg.reshape(_L, 1, _H)
    bc12 = bc1.reshape(1, _H)
    bc22 = bc2.reshape(1, _G)
    Wc3p = jnp.pad(Wc3, ((0, 0), (0, _H - _C)))
    bc3p = jnp.pad(bc3, (0, _H - _C)).reshape(1, _H)

    degp = _sc_degree(dst_chunks, ones_rows, zeros_rows)
    dinv, h, y = _tc_prologue(xp, W_in, b_in2, Wg[0], degp)
    for i in range(_L):
        p = _sc_msgpass(y, src_chunks, dst_chunks, zeros_rows)
        if i < _L - 1:
            h, y = _tc_combine(p, y, h, dinv, bg2[i], Wg[i + 1], True)
        else:
            (h,) = _tc_combine(p, y, h, dinv, bg2[i], Wg[i], False)

    out = _tc_pool_mlp(h, batch_p, Wc1, bc12, Wc2, bc22, Wc3p, bc3p)
    return out[:, :_C]


# speedup vs baseline: 19.0867x; 1.2870x over previous
import functools

import jax
import jax.numpy as jnp
from jax import lax
from jax.experimental import pallas as pl
from jax.experimental.pallas import tpu as pltpu
from jax.experimental.pallas import tpu_sc as plsc

_N, _E, _H, _G, _C, _L = 10000, 320000, 128, 64, 5, 4
_NC, _NS = 2, 16
_NW = _NC * _NS
_K = 128
_GB = 8
_CH = (-(-_E // (_NW * _K * _GB))) * _GB
_NG = _CH // _GB
_EPT = _CH * _K
_EP = _EPT * _NW
_NPAD = 10240
_RPS = _NPAD // _NS
_DW = 128

@functools.cache
def _mesh():
    return plsc.VectorSubcoreMesh(core_axis_name="c", subcore_axis_name="s")


def _sc_degree(dst_chunks, ones_rows, zeros_rows):
    @functools.partial(
        pl.kernel,
        out_type=jax.ShapeDtypeStruct((_NC, _NPAD, _DW), jnp.float32),
        mesh=_mesh(),
        scratch_types=[
            pltpu.VMEM((_CH, _K), jnp.int32),
            pltpu.VMEM((_K, _DW), jnp.float32),
            pltpu.VMEM_SHARED((_NPAD, _DW), jnp.float32),
        ],
    )
    def k(dst_hbm, ones_hbm, zer_hbm, out_hbm, idx_v, ones_v, acc):
        c = lax.axis_index("c")
        s = lax.axis_index("s")
        wid = c * _NS + s
        pltpu.sync_copy(dst_hbm.at[wid], idx_v)
        pltpu.sync_copy(ones_hbm, ones_v)
        pltpu.sync_copy(zer_hbm, acc.at[pl.ds(s * _RPS, _RPS)])
        plsc.subcore_barrier()

        @pl.loop(0, _CH)
        def _(ch):
            pltpu.sync_copy(ones_v, acc.at[idx_v.at[ch]], add=True)

        plsc.subcore_barrier()
        pltpu.sync_copy(acc.at[pl.ds(s * _RPS, _RPS)],
                        out_hbm.at[c, pl.ds(s * _RPS, _RPS)])

    return k(dst_chunks, ones_rows, zeros_rows)


def _sc_msgpass(y, src_chunks, dst_chunks, zeros_rows):
    @functools.partial(
        pl.kernel,
        out_type=jax.ShapeDtypeStruct((_NC, _NPAD, _H), jnp.float32),
        mesh=_mesh(),
        scratch_types=[
            pltpu.VMEM((_GB, _K), jnp.int32),
            pltpu.VMEM((_GB, _K), jnp.int32),
            pltpu.VMEM((_CH, _K), jnp.int32),
            pltpu.VMEM((_K, _H), jnp.float32),
            pltpu.VMEM((_K, _H), jnp.float32),
            pltpu.VMEM_SHARED((_NPAD, _H), jnp.float32),
            pltpu.SemaphoreType.DMA,
            pltpu.SemaphoreType.DMA,
            pltpu.SemaphoreType.DMA,
        ],
    )
    def k(y_hbm, src_hbm, dst_hbm, zer_hbm, out_hbm,
          sgrp0, sgrp1, dst_v, rows0, rows1, acc, semA, semB, semI):
        c = lax.axis_index("c")
        s = lax.axis_index("s")
        wid = c * _NS + s
        pltpu.sync_copy(dst_hbm.at[wid], dst_v)
        pltpu.sync_copy(src_hbm.at[wid, pl.ds(0, _GB)], sgrp0)
        pltpu.sync_copy(zer_hbm, acc.at[pl.ds(s * _RPS, _RPS)])
        plsc.subcore_barrier()

        rows = (rows0, rows1)
        sems = (semA, semB)

        def process(sgrp, base):
            pltpu.async_copy(y_hbm.at[sgrp.at[0]], rows0, semA)
            for j in range(_GB):
                if j + 1 < _GB:
                    pltpu.async_copy(y_hbm.at[sgrp.at[j + 1]],
                                     rows[(j + 1) % 2], sems[(j + 1) % 2])
                pltpu.make_async_copy(y_hbm.at[sgrp.at[j]], rows[j % 2],
                                      sems[j % 2]).wait()
                pltpu.sync_copy(rows[j % 2], acc.at[dst_v.at[base + j]],
                                add=True)

        @pl.loop(0, _NG, step=2)
        def _(g):
            @pl.when(g > 0)
            def _():
                pltpu.make_async_copy(
                    src_hbm.at[wid, pl.ds(g * _GB, _GB)], sgrp0, semI).wait()

            pltpu.async_copy(src_hbm.at[wid, pl.ds((g + 1) * _GB, _GB)],
                             sgrp1, semI)
            process(sgrp0, g * _GB)
            pltpu.make_async_copy(src_hbm.at[wid, pl.ds((g + 1) * _GB, _GB)],
                                  sgrp1, semI).wait()

            @pl.when(g + 2 < _NG)
            def _():
                pltpu.async_copy(src_hbm.at[wid, pl.ds((g + 2) * _GB, _GB)],
                                 sgrp0, semI)

            process(sgrp1, (g + 1) * _GB)

        plsc.subcore_barrier()
        pltpu.sync_copy(acc.at[pl.ds(s * _RPS, _RPS)],
                        out_hbm.at[c, pl.ds(s * _RPS, _RPS)])

    return k(y, src_chunks, dst_chunks, zeros_rows)


def _tc_prologue(xp, W_in, b_in, Wg0, degp):
    br = 1280
    nb = _NPAD // br

    def body(x_ref, w_ref, b_ref, wg_ref, degp_ref, dinv_ref, h0_ref, y0_ref):
        deg = 1.0 + degp_ref[0, :, 0:1] + degp_ref[1, :, 0:1]
        dinv = lax.rsqrt(deg)
        dinv_ref[...] = dinv
        h0 = jnp.dot(x_ref[...], w_ref[...],
                     preferred_element_type=jnp.float32) + b_ref[...]
        h0_ref[...] = h0
        y0_ref[...] = jnp.dot(h0, wg_ref[...],
                              preferred_element_type=jnp.float32) * dinv

    return pl.pallas_call(
        body,
        grid=(nb,),
        in_specs=[
            pl.BlockSpec((br, _H), lambda i: (i, 0)),
            pl.BlockSpec((_H, _H), lambda i: (0, 0)),
            pl.BlockSpec((1, _H), lambda i: (0, 0)),
            pl.BlockSpec((_H, _H), lambda i: (0, 0)),
            pl.BlockSpec((_NC, br, _DW), lambda i: (0, i, 0)),
        ],
        out_specs=[
            pl.BlockSpec((br, 1), lambda i: (i, 0)),
            pl.BlockSpec((br, _H), lambda i: (i, 0)),
            pl.BlockSpec((br, _H), lambda i: (i, 0)),
        ],
        out_shape=[
            jax.ShapeDtypeStruct((_NPAD, 1), jnp.float32),
            jax.ShapeDtypeStruct((_NPAD, _H), jnp.float32),
            jax.ShapeDtypeStruct((_NPAD, _H), jnp.float32),
        ],
    )(xp, W_in, b_in, Wg0, degp)


def _tc_combine(p, y, h, dinv, bg_i, Wg_next, project):
    br = 1280
    nb = _NPAD // br

    def body(p_ref, y_ref, h_ref, dinv_ref, bg_ref, wg_ref, *outs):
        dinv = dinv_ref[...]
        agg = p_ref[0] + p_ref[1] + y_ref[...]
        hn = jnp.maximum(dinv * agg + bg_ref[...] + h_ref[...], 0.0)
        outs[0][...] = hn
        if project:
            outs[1][...] = jnp.dot(hn, wg_ref[...],
                                   preferred_element_type=jnp.float32) * dinv

    out_specs = [pl.BlockSpec((br, _H), lambda i: (i, 0))]
    out_shape = [jax.ShapeDtypeStruct((_NPAD, _H), jnp.float32)]
    if project:
        out_specs = out_specs * 2
        out_shape = out_shape * 2
    return pl.pallas_call(
        body,
        grid=(nb,),
        in_specs=[
            pl.BlockSpec((_NC, br, _H), lambda i: (0, i, 0)),
            pl.BlockSpec((br, _H), lambda i: (i, 0)),
            pl.BlockSpec((br, _H), lambda i: (i, 0)),
            pl.BlockSpec((br, 1), lambda i: (i, 0)),
            pl.BlockSpec((1, _H), lambda i: (0, 0)),
            pl.BlockSpec((_H, _H), lambda i: (0, 0)),
        ],
        out_specs=out_specs,
        out_shape=out_shape,
    )(p, y, h, dinv, bg_i, Wg_next)


def _tc_pool_mlp(h, batch_p, Wc1, bc1, Wc2, bc2, Wc3p, bc3p):
    br = 1280
    nb = _NPAD // br
    neg_inf = float("-inf")

    def body(h_ref, b_ref, wc1_ref, bc1_ref, wc2_ref, bc2_ref, wc3_ref,
             bc3_ref, out_ref, sums, cnts, mx):
        i = pl.program_id(0)

        @pl.when(i == 0)
        def _():
            sums[...] = jnp.zeros((_G, _H), jnp.float32)
            cnts[...] = jnp.zeros((_G, _H), jnp.float32)
            mx[...] = jnp.full((_G, _H), neg_inf, jnp.float32)

        hb = h_ref[...]
        bb = b_ref[...]
        gids = lax.broadcasted_iota(jnp.int32, (1, _G), 1)
        m = (bb == gids).astype(jnp.float32)
        dn = (((0,), (0,)), ((), ()))
        sums[...] += lax.dot_general(m, hb, dn,
                                     preferred_element_type=jnp.float32)
        cnts[...] += lax.dot_general(m, jnp.ones_like(hb), dn,
                                     preferred_element_type=jnp.float32)
        seg = [jnp.max(jnp.where(bb == g, hb, neg_inf), axis=0)[None, :]
               for g in range(_G)]
        mx[...] = jnp.maximum(mx[...], jnp.concatenate(seg, axis=0))

        @pl.when(i == nb - 1)
        def _():
            mean = sums[...] / jnp.maximum(cnts[...], 1.0)
            pooled = jnp.concatenate([mean, mx[...]], axis=1)
            z = jnp.dot(pooled, wc1_ref[...],
                        preferred_element_type=jnp.float32) + bc1_ref[...]
            z = jnp.maximum(z, 0.0)
            z = jnp.dot(z, wc2_ref[...],
                        preferred_element_type=jnp.float32) + bc2_ref[...]
            z = jnp.maximum(z, 0.0)
            out_ref[...] = jnp.dot(z, wc3_ref[...],
                                   preferred_element_type=jnp.float32) + bc3_ref[...]

    return pl.pallas_call(
        body,
        grid=(nb,),
        in_specs=[
            pl.BlockSpec((br, _H), lambda i: (i, 0)),
            pl.BlockSpec((br, 1), lambda i: (i, 0)),
            pl.BlockSpec((2 * _H, _H), lambda i: (0, 0)),
            pl.BlockSpec((1, _H), lambda i: (0, 0)),
            pl.BlockSpec((_H, _G), lambda i: (0, 0)),
            pl.BlockSpec((1, _G), lambda i: (0, 0)),
            pl.BlockSpec((_G, _H), lambda i: (0, 0)),
            pl.BlockSpec((1, _H), lambda i: (0, 0)),
        ],
        out_specs=pl.BlockSpec((_G, _H), lambda i: (0, 0)),
        out_shape=jax.ShapeDtypeStruct((_G, _H), jnp.float32),
        scratch_shapes=[
            pltpu.VMEM((_G, _H), jnp.float32),
            pltpu.VMEM((_G, _H), jnp.float32),
            pltpu.VMEM((_G, _H), jnp.float32),
        ],
    )(h, batch_p, Wc1, bc1, Wc2, bc2, Wc3p, bc3p)


def kernel(x, edge_index, batch, W_in, b_in, Wg, bg, Wc1, bc1, Wc2, bc2,
           Wc3, bc3):
    src = edge_index[0]
    dst = edge_index[1]
    npad_rows = _NPAD - _N
    pad_e = _EP - _E
    pad_src = (jnp.arange(pad_e, dtype=jnp.int32) % 1024)
    pad_dst = _N + (jnp.arange(pad_e, dtype=jnp.int32) % npad_rows)
    src_chunks = jnp.concatenate([src, pad_src]).reshape(_NW, _CH, _K)
    dst_chunks = jnp.concatenate([dst, pad_dst]).reshape(_NW, _CH, _K)

    ones_rows = jnp.ones((_K, _DW), jnp.float32)
    zeros_rows = jnp.zeros((_RPS, _H), jnp.float32)

    xp = jnp.pad(x, ((0, npad_rows), (0, 0)))
    batch_p = jnp.pad(batch, (0, npad_rows), constant_values=_G)
    batch_p = batch_p.reshape(_NPAD, 1)

    b_in2 = b_in.reshape(1, _H)
    bg2 = bg.reshape(_L, 1, _H)
    bc12 = bc1.reshape(1, _H)
    bc22 = bc2.reshape(1, _G)
    Wc3p = jnp.pad(Wc3, ((0, 0), (0, _H - _C)))
    bc3p = jnp.pad(bc3, (0, _H - _C)).reshape(1, _H)

    degp = _sc_degree(dst_chunks, ones_rows, zeros_rows)
    dinv, h, y = _tc_prologue(xp, W_in, b_in2, Wg[0], degp)
    for i in range(_L):
        p = _sc_msgpass(y, src_chunks, dst_chunks, zeros_rows)
        if i < _L - 1:
            h, y = _tc_combine(p, y, h, dinv, bg2[i], Wg[i + 1], True)
        else:
            (h,) = _tc_combine(p, y, h, dinv, bg2[i], Wg[i], False)

    out = _tc_pool_mlp(h, batch_p, Wc1, bc12, Wc2, bc22, Wc3p, bc3p)
    return out[:, :_C]


# speedup vs baseline: 20.7784x; 1.0886x over previous
import functools

import jax
import jax.numpy as jnp
from jax import lax
from jax.experimental import pallas as pl
from jax.experimental.pallas import tpu as pltpu
from jax.experimental.pallas import tpu_sc as plsc

_N, _E, _H, _G, _C, _L = 10000, 320000, 128, 64, 5, 4
_NC, _NS = 2, 16
_NW = _NC * _NS
_K = 128
_GB = 8
_CH = (-(-_E // (_NW * _K * _GB))) * _GB
_NG = _CH // _GB
_EPT = _CH * _K
_EP = _EPT * _NW
_NPAD = 10240
_RPS = _NPAD // _NS
_DW = 128

@functools.cache
def _mesh():
    return plsc.VectorSubcoreMesh(core_axis_name="c", subcore_axis_name="s")


def _sc_degree(dst_chunks, ones_rows, zeros_rows):
    @functools.partial(
        pl.kernel,
        out_type=jax.ShapeDtypeStruct((_NC, _NPAD, _DW), jnp.float32),
        mesh=_mesh(),
        scratch_types=[
            pltpu.VMEM((_CH, _K), jnp.int32),
            pltpu.VMEM((_K, _DW), jnp.float32),
            pltpu.VMEM_SHARED((_NPAD, _DW), jnp.float32),
        ],
    )
    def k(dst_hbm, ones_hbm, zer_hbm, out_hbm, idx_v, ones_v, acc):
        c = lax.axis_index("c")
        s = lax.axis_index("s")
        wid = c * _NS + s
        pltpu.sync_copy(dst_hbm.at[wid], idx_v)
        pltpu.sync_copy(ones_hbm, ones_v)
        pltpu.sync_copy(zer_hbm, acc.at[pl.ds(s * _RPS, _RPS)])
        plsc.subcore_barrier()

        @pl.loop(0, _CH)
        def _(ch):
            pltpu.sync_copy(ones_v, acc.at[idx_v.at[ch]], add=True)

        plsc.subcore_barrier()
        pltpu.sync_copy(acc.at[pl.ds(s * _RPS, _RPS)],
                        out_hbm.at[c, pl.ds(s * _RPS, _RPS)])

    return k(dst_chunks, ones_rows, zeros_rows)


def _sc_msgpass(y, src_chunks, dst_chunks, zeros_rows):
    @functools.partial(
        pl.kernel,
        out_type=jax.ShapeDtypeStruct((_NC, _NPAD, _H), jnp.float32),
        mesh=_mesh(),
        scratch_types=[
            pltpu.VMEM((_GB, _K), jnp.int32),
            pltpu.VMEM((_GB, _K), jnp.int32),
            pltpu.VMEM((_CH, _K), jnp.int32),
            pltpu.VMEM((_K, _H), jnp.float32),
            pltpu.VMEM((_K, _H), jnp.float32),
            pltpu.VMEM_SHARED((_NPAD, _H), jnp.float32),
            pltpu.SemaphoreType.DMA,
            pltpu.SemaphoreType.DMA,
            pltpu.SemaphoreType.DMA,
        ],
    )
    def k(y_hbm, src_hbm, dst_hbm, zer_hbm, out_hbm,
          sgrp0, sgrp1, dst_v, rows0, rows1, acc, semA, semB, semI):
        c = lax.axis_index("c")
        s = lax.axis_index("s")
        wid = c * _NS + s
        pltpu.sync_copy(dst_hbm.at[wid], dst_v)
        pltpu.sync_copy(src_hbm.at[wid, pl.ds(0, _GB)], sgrp0)
        pltpu.sync_copy(zer_hbm, acc.at[pl.ds(s * _RPS, _RPS)])
        plsc.subcore_barrier()

        rows = (rows0, rows1)
        sems = (semA, semB)

        def process(sgrp, base):
            pltpu.async_copy(y_hbm.at[sgrp.at[0]], rows0, semA)
            for j in range(_GB):
                if j + 1 < _GB:
                    pltpu.async_copy(y_hbm.at[sgrp.at[j + 1]],
                                     rows[(j + 1) % 2], sems[(j + 1) % 2])
                pltpu.make_async_copy(y_hbm.at[sgrp.at[j]], rows[j % 2],
                                      sems[j % 2]).wait()
                pltpu.sync_copy(rows[j % 2], acc.at[dst_v.at[base + j]],
                                add=True)

        @pl.loop(0, _NG, step=2)
        def _(g):
            @pl.when(g > 0)
            def _():
                pltpu.make_async_copy(
                    src_hbm.at[wid, pl.ds(g * _GB, _GB)], sgrp0, semI).wait()

            pltpu.async_copy(src_hbm.at[wid, pl.ds((g + 1) * _GB, _GB)],
                             sgrp1, semI)
            process(sgrp0, g * _GB)
            pltpu.make_async_copy(src_hbm.at[wid, pl.ds((g + 1) * _GB, _GB)],
                                  sgrp1, semI).wait()

            @pl.when(g + 2 < _NG)
            def _():
                pltpu.async_copy(src_hbm.at[wid, pl.ds((g + 2) * _GB, _GB)],
                                 sgrp0, semI)

            process(sgrp1, (g + 1) * _GB)

        plsc.subcore_barrier()
        pltpu.sync_copy(acc.at[pl.ds(s * _RPS, _RPS)],
                        out_hbm.at[c, pl.ds(s * _RPS, _RPS)])

    return k(y, src_chunks, dst_chunks, zeros_rows)


def _tc_prologue(xp, W_in, b_in, Wg0, degp):
    br = 1280
    nb = _NPAD // br

    def body(x_ref, w_ref, b_ref, wg_ref, degp_ref, dinv_ref, h0_ref, y0_ref):
        deg = 1.0 + degp_ref[0, :, 0:1] + degp_ref[1, :, 0:1]
        dinv = lax.rsqrt(deg)
        dinv_ref[...] = dinv
        h0 = jnp.dot(x_ref[...], w_ref[...],
                     preferred_element_type=jnp.float32) + b_ref[...]
        h0_ref[...] = h0
        y0_ref[...] = jnp.dot(h0, wg_ref[...],
                              preferred_element_type=jnp.float32) * dinv

    return pl.pallas_call(
        body,
        grid=(nb,),
        in_specs=[
            pl.BlockSpec((br, _H), lambda i: (i, 0)),
            pl.BlockSpec((_H, _H), lambda i: (0, 0)),
            pl.BlockSpec((1, _H), lambda i: (0, 0)),
            pl.BlockSpec((_H, _H), lambda i: (0, 0)),
            pl.BlockSpec((_NC, br, _DW), lambda i: (0, i, 0)),
        ],
        out_specs=[
            pl.BlockSpec((br, 1), lambda i: (i, 0)),
            pl.BlockSpec((br, _H), lambda i: (i, 0)),
            pl.BlockSpec((br, _H), lambda i: (i, 0)),
        ],
        out_shape=[
            jax.ShapeDtypeStruct((_NPAD, 1), jnp.float32),
            jax.ShapeDtypeStruct((_NPAD, _H), jnp.float32),
            jax.ShapeDtypeStruct((_NPAD, _H), jnp.float32),
        ],
    )(xp, W_in, b_in, Wg0, degp)


def _tc_combine(p, y, h, dinv, bg_i, Wg_next, project):
    br = 1280
    nb = _NPAD // br

    def body(p_ref, y_ref, h_ref, dinv_ref, bg_ref, wg_ref, *outs):
        dinv = dinv_ref[...]
        agg = p_ref[0] + p_ref[1] + y_ref[...]
        hn = jnp.maximum(dinv * agg + bg_ref[...] + h_ref[...], 0.0)
        outs[0][...] = hn
        if project:
            outs[1][...] = jnp.dot(hn, wg_ref[...],
                                   preferred_element_type=jnp.float32) * dinv

    out_specs = [pl.BlockSpec((br, _H), lambda i: (i, 0))]
    out_shape = [jax.ShapeDtypeStruct((_NPAD, _H), jnp.float32)]
    if project:
        out_specs = out_specs * 2
        out_shape = out_shape * 2
    return pl.pallas_call(
        body,
        grid=(nb,),
        in_specs=[
            pl.BlockSpec((_NC, br, _H), lambda i: (0, i, 0)),
            pl.BlockSpec((br, _H), lambda i: (i, 0)),
            pl.BlockSpec((br, _H), lambda i: (i, 0)),
            pl.BlockSpec((br, 1), lambda i: (i, 0)),
            pl.BlockSpec((1, _H), lambda i: (0, 0)),
            pl.BlockSpec((_H, _H), lambda i: (0, 0)),
        ],
        out_specs=out_specs,
        out_shape=out_shape,
    )(p, y, h, dinv, bg_i, Wg_next)


_GT = 72
_SPT = _NPAD // _NW


def _sc_pool(h, batch_p1d, neg_table, zero_table):
    @functools.partial(
        pl.kernel,
        out_type=(jax.ShapeDtypeStruct((_NW, _GT, _H), jnp.float32),
                  jax.ShapeDtypeStruct((_NW, _GT, _H), jnp.float32),
                  jax.ShapeDtypeStruct((_NW, _GT, _H), jnp.float32)),
        mesh=_mesh(),
        compiler_params=pltpu.CompilerParams(needs_layout_passes=False),
        scratch_types=[
            pltpu.VMEM((_SPT, _H), jnp.float32),
            pltpu.VMEM((_SPT,), jnp.int32),
            pltpu.VMEM((_GT, _H), jnp.float32),
            pltpu.VMEM((_GT, _H), jnp.float32),
            pltpu.VMEM((_GT, _H), jnp.float32),
        ],
    )
    def k(h_hbm, b_hbm, negt_hbm, zerot_hbm, omax_hbm, osum_hbm, ocnt_hbm,
          strip_v, gid_v, mx_v, sm_v, ct_v):
        c = lax.axis_index("c")
        s = lax.axis_index("s")
        wid = c * _NS + s
        base = wid * _SPT
        pltpu.sync_copy(h_hbm.at[pl.ds(base, _SPT)], strip_v)
        pltpu.sync_copy(b_hbm.at[pl.ds(base, _SPT)], gid_v)
        pltpu.sync_copy(negt_hbm, mx_v)
        pltpu.sync_copy(zerot_hbm, sm_v)
        pltpu.sync_copy(zerot_hbm, ct_v)
        lanes = lax.broadcasted_iota(jnp.int32, (16,), 0)
        lane0 = lanes == 0
        onev = jnp.ones((16,), jnp.float32)

        @pl.loop(0, _SPT, init_carry=jnp.zeros((16,), jnp.int32))
        def _(i, ivec):
            gidv = plsc.load_gather(gid_v, [ivec])
            plsc.addupdate_scatter(ct_v, [gidv, lanes], onev, mask=lane0)
            for kk in range(_H // 16):
                col = kk * 16 + lanes
                row = plsc.load_gather(strip_v, [ivec, col])
                cur = plsc.load_gather(mx_v, [gidv, col])
                plsc.store_scatter(mx_v, [gidv, col], jnp.maximum(cur, row))
                plsc.addupdate_scatter(sm_v, [gidv, col], row)
            return ivec + 1

        pltpu.sync_copy(mx_v, omax_hbm.at[wid])
        pltpu.sync_copy(sm_v, osum_hbm.at[wid])
        pltpu.sync_copy(ct_v, ocnt_hbm.at[wid])

    return k(h, batch_p1d, neg_table, zero_table)


def _tc_pool_reduce_mlp(maxs, sums, cnts, Wc1, bc1, Wc2, bc2, Wc3p, bc3p):
    def body(mx_ref, sm_ref, ct_ref, wc1_ref, bc1_ref, wc2_ref, bc2_ref,
             wc3_ref, bc3_ref, out_ref):
        mx = jnp.max(mx_ref[...], axis=0)[:_G]
        sm = jnp.sum(sm_ref[...], axis=0)[:_G]
        cnt = jnp.sum(ct_ref[...], axis=0)[:_G, 0:1]
        mean = sm / jnp.maximum(cnt, 1.0)
        pooled = jnp.concatenate([mean, mx], axis=1)
        z = jnp.dot(pooled, wc1_ref[...],
                    preferred_element_type=jnp.float32) + bc1_ref[...]
        z = jnp.maximum(z, 0.0)
        z = jnp.dot(z, wc2_ref[...],
                    preferred_element_type=jnp.float32) + bc2_ref[...]
        z = jnp.maximum(z, 0.0)
        out_ref[...] = jnp.dot(z, wc3_ref[...],
                               preferred_element_type=jnp.float32) + bc3_ref[...]

    return pl.pallas_call(
        body,
        out_shape=jax.ShapeDtypeStruct((_G, _H), jnp.float32),
    )(maxs, sums, cnts, Wc1, bc1, Wc2, bc2, Wc3p, bc3p)


def kernel(x, edge_index, batch, W_in, b_in, Wg, bg, Wc1, bc1, Wc2, bc2,
           Wc3, bc3):
    src = edge_index[0]
    dst = edge_index[1]
    npad_rows = _NPAD - _N
    pad_e = _EP - _E
    pad_src = (jnp.arange(pad_e, dtype=jnp.int32) % 1024)
    pad_dst = _N + (jnp.arange(pad_e, dtype=jnp.int32) % npad_rows)
    src_chunks = jnp.concatenate([src, pad_src]).reshape(_NW, _CH, _K)
    dst_chunks = jnp.concatenate([dst, pad_dst]).reshape(_NW, _CH, _K)

    ones_rows = jnp.ones((_K, _DW), jnp.float32)
    zeros_rows = jnp.zeros((_RPS, _H), jnp.float32)

    xp = jnp.pad(x, ((0, npad_rows), (0, 0)))
    batch_p1d = jnp.pad(batch, (0, npad_rows), constant_values=_G)
    neg_table = jnp.full((_GT, _H), float("-inf"), jnp.float32)
    zero_table = jnp.zeros((_GT, _H), jnp.float32)

    b_in2 = b_in.reshape(1, _H)
    bg2 = bg.reshape(_L, 1, _H)
    bc12 = bc1.reshape(1, _H)
    bc22 = bc2.reshape(1, _G)
    Wc3p = jnp.pad(Wc3, ((0, 0), (0, _H - _C)))
    bc3p = jnp.pad(bc3, (0, _H - _C)).reshape(1, _H)

    degp = _sc_degree(dst_chunks, ones_rows, zeros_rows)
    dinv, h, y = _tc_prologue(xp, W_in, b_in2, Wg[0], degp)
    for i in range(_L):
        p = _sc_msgpass(y, src_chunks, dst_chunks, zeros_rows)
        if i < _L - 1:
            h, y = _tc_combine(p, y, h, dinv, bg2[i], Wg[i + 1], True)
        else:
            (h,) = _tc_combine(p, y, h, dinv, bg2[i], Wg[i], False)

    maxs, sums, cnts = _sc_pool(h, batch_p1d, neg_table, zero_table)
    out = _tc_pool_reduce_mlp(maxs, sums, cnts, Wc1, bc12, Wc2, bc22, Wc3p,
                              bc3p)
    return out[:, :_C]


# speedup vs baseline: 21.5807x; 1.0386x over previous
import functools

import jax
import jax.numpy as jnp
from jax import lax
from jax.experimental import pallas as pl
from jax.experimental.pallas import tpu as pltpu
from jax.experimental.pallas import tpu_sc as plsc

_N, _E, _H, _G, _C, _L = 10000, 320000, 128, 64, 5, 4
_NC, _NS = 2, 16
_NW = _NC * _NS
_K = 128
_GB = 16
_CH = (-(-_E // (_NW * _K * _GB))) * _GB
_NG = _CH // _GB
_EPT = _CH * _K
_EP = _EPT * _NW
_NPAD = 10240
_RPS = _NPAD // _NS
_DW = 128

@functools.cache
def _mesh():
    return plsc.VectorSubcoreMesh(core_axis_name="c", subcore_axis_name="s")


def _sc_degree(dst_chunks, ones_rows, zeros_rows):
    @functools.partial(
        pl.kernel,
        out_type=jax.ShapeDtypeStruct((_NC, _NPAD, _DW), jnp.float32),
        mesh=_mesh(),
        scratch_types=[
            pltpu.VMEM((_CH, _K), jnp.int32),
            pltpu.VMEM((_K, _DW), jnp.float32),
            pltpu.VMEM_SHARED((_NPAD, _DW), jnp.float32),
        ],
    )
    def k(dst_hbm, ones_hbm, zer_hbm, out_hbm, idx_v, ones_v, acc):
        c = lax.axis_index("c")
        s = lax.axis_index("s")
        wid = c * _NS + s
        pltpu.sync_copy(dst_hbm.at[wid], idx_v)
        pltpu.sync_copy(ones_hbm, ones_v)
        pltpu.sync_copy(zer_hbm, acc.at[pl.ds(s * _RPS, _RPS)])
        plsc.subcore_barrier()

        @pl.loop(0, _CH)
        def _(ch):
            pltpu.sync_copy(ones_v, acc.at[idx_v.at[ch]], add=True)

        plsc.subcore_barrier()
        pltpu.sync_copy(acc.at[pl.ds(s * _RPS, _RPS)],
                        out_hbm.at[c, pl.ds(s * _RPS, _RPS)])

    return k(dst_chunks, ones_rows, zeros_rows)


def _sc_msgpass(y, src_chunks, dst_chunks, zeros_rows):
    @functools.partial(
        pl.kernel,
        out_type=jax.ShapeDtypeStruct((_NC, _NPAD, _H), jnp.float32),
        mesh=_mesh(),
        scratch_types=[
            pltpu.VMEM((_GB, _K), jnp.int32),
            pltpu.VMEM((_GB, _K), jnp.int32),
            pltpu.VMEM((_CH, _K), jnp.int32),
            pltpu.VMEM((_K, _H), jnp.float32),
            pltpu.VMEM((_K, _H), jnp.float32),
            pltpu.VMEM_SHARED((_NPAD, _H), jnp.float32),
            pltpu.SemaphoreType.DMA,
            pltpu.SemaphoreType.DMA,
            pltpu.SemaphoreType.DMA,
        ],
    )
    def k(y_hbm, src_hbm, dst_hbm, zer_hbm, out_hbm,
          sgrp0, sgrp1, dst_v, rows0, rows1, acc, semA, semB, semI):
        c = lax.axis_index("c")
        s = lax.axis_index("s")
        wid = c * _NS + s
        pltpu.sync_copy(dst_hbm.at[wid], dst_v)
        pltpu.sync_copy(src_hbm.at[wid, pl.ds(0, _GB)], sgrp0)
        pltpu.sync_copy(zer_hbm, acc.at[pl.ds(s * _RPS, _RPS)])
        plsc.subcore_barrier()

        rows = (rows0, rows1)
        sems = (semA, semB)

        def process(sgrp, base):
            pltpu.async_copy(y_hbm.at[sgrp.at[0]], rows0, semA)
            for j in range(_GB):
                if j + 1 < _GB:
                    pltpu.async_copy(y_hbm.at[sgrp.at[j + 1]],
                                     rows[(j + 1) % 2], sems[(j + 1) % 2])
                pltpu.make_async_copy(y_hbm.at[sgrp.at[j]], rows[j % 2],
                                      sems[j % 2]).wait()
                pltpu.sync_copy(rows[j % 2], acc.at[dst_v.at[base + j]],
                                add=True)

        @pl.loop(0, _NG - (_NG % 2), step=2)
        def _(g):
            @pl.when(g > 0)
            def _():
                pltpu.make_async_copy(
                    src_hbm.at[wid, pl.ds(g * _GB, _GB)], sgrp0, semI).wait()

            pltpu.async_copy(src_hbm.at[wid, pl.ds((g + 1) * _GB, _GB)],
                             sgrp1, semI)
            process(sgrp0, g * _GB)
            pltpu.make_async_copy(src_hbm.at[wid, pl.ds((g + 1) * _GB, _GB)],
                                  sgrp1, semI).wait()

            @pl.when(g + 2 < _NG)
            def _():
                pltpu.async_copy(src_hbm.at[wid, pl.ds((g + 2) * _GB, _GB)],
                                 sgrp0, semI)

            process(sgrp1, (g + 1) * _GB)

        if _NG % 2:
            pltpu.make_async_copy(
                src_hbm.at[wid, pl.ds((_NG - 1) * _GB, _GB)], sgrp0,
                semI).wait()
            process(sgrp0, (_NG - 1) * _GB)

        plsc.subcore_barrier()
        pltpu.sync_copy(acc.at[pl.ds(s * _RPS, _RPS)],
                        out_hbm.at[c, pl.ds(s * _RPS, _RPS)])

    return k(y, src_chunks, dst_chunks, zeros_rows)


def _tc_prologue(xp, W_in, b_in, Wg0, degp):
    br = 1280
    nb = _NPAD // br

    def body(x_ref, w_ref, b_ref, wg_ref, degp_ref, dinv_ref, h0_ref, y0_ref):
        deg = 1.0 + degp_ref[0, :, 0:1] + degp_ref[1, :, 0:1]
        dinv = lax.rsqrt(deg)
        dinv_ref[...] = dinv
        h0 = jnp.dot(x_ref[...], w_ref[...],
                     preferred_element_type=jnp.float32) + b_ref[...]
        h0_ref[...] = h0
        y0_ref[...] = jnp.dot(h0, wg_ref[...],
                              preferred_element_type=jnp.float32) * dinv

    return pl.pallas_call(
        body,
        grid=(nb,),
        in_specs=[
            pl.BlockSpec((br, _H), lambda i: (i, 0)),
            pl.BlockSpec((_H, _H), lambda i: (0, 0)),
            pl.BlockSpec((1, _H), lambda i: (0, 0)),
            pl.BlockSpec((_H, _H), lambda i: (0, 0)),
            pl.BlockSpec((_NC, br, _DW), lambda i: (0, i, 0)),
        ],
        out_specs=[
            pl.BlockSpec((br, 1), lambda i: (i, 0)),
            pl.BlockSpec((br, _H), lambda i: (i, 0)),
            pl.BlockSpec((br, _H), lambda i: (i, 0)),
        ],
        out_shape=[
            jax.ShapeDtypeStruct((_NPAD, 1), jnp.float32),
            jax.ShapeDtypeStruct((_NPAD, _H), jnp.float32),
            jax.ShapeDtypeStruct((_NPAD, _H), jnp.float32),
        ],
    )(xp, W_in, b_in, Wg0, degp)


def _tc_combine(p, y, h, dinv, bg_i, Wg_next, project):
    br = 1280
    nb = _NPAD // br

    def body(p_ref, y_ref, h_ref, dinv_ref, bg_ref, wg_ref, *outs):
        dinv = dinv_ref[...]
        agg = p_ref[0] + p_ref[1] + y_ref[...]
        hn = jnp.maximum(dinv * agg + bg_ref[...] + h_ref[...], 0.0)
        outs[0][...] = hn
        if project:
            outs[1][...] = jnp.dot(hn, wg_ref[...],
                                   preferred_element_type=jnp.float32) * dinv

    out_specs = [pl.BlockSpec((br, _H), lambda i: (i, 0))]
    out_shape = [jax.ShapeDtypeStruct((_NPAD, _H), jnp.float32)]
    if project:
        out_specs = out_specs * 2
        out_shape = out_shape * 2
    return pl.pallas_call(
        body,
        grid=(nb,),
        in_specs=[
            pl.BlockSpec((_NC, br, _H), lambda i: (0, i, 0)),
            pl.BlockSpec((br, _H), lambda i: (i, 0)),
            pl.BlockSpec((br, _H), lambda i: (i, 0)),
            pl.BlockSpec((br, 1), lambda i: (i, 0)),
            pl.BlockSpec((1, _H), lambda i: (0, 0)),
            pl.BlockSpec((_H, _H), lambda i: (0, 0)),
        ],
        out_specs=out_specs,
        out_shape=out_shape,
    )(p, y, h, dinv, bg_i, Wg_next)


_GT = 72
_SPT = _NPAD // _NW


def _sc_pool(h, batch_p1d, neg_table, zero_table):
    @functools.partial(
        pl.kernel,
        out_type=(jax.ShapeDtypeStruct((_NW, _GT, _H), jnp.float32),
                  jax.ShapeDtypeStruct((_NW, _GT, _H), jnp.float32),
                  jax.ShapeDtypeStruct((_NW, _GT, _H), jnp.float32)),
        mesh=_mesh(),
        compiler_params=pltpu.CompilerParams(needs_layout_passes=False),
        scratch_types=[
            pltpu.VMEM((_SPT, _H), jnp.float32),
            pltpu.VMEM((_SPT,), jnp.int32),
            pltpu.VMEM((_GT, _H), jnp.float32),
            pltpu.VMEM((_GT, _H), jnp.float32),
            pltpu.VMEM((_GT, _H), jnp.float32),
        ],
    )
    def k(h_hbm, b_hbm, negt_hbm, zerot_hbm, omax_hbm, osum_hbm, ocnt_hbm,
          strip_v, gid_v, mx_v, sm_v, ct_v):
        c = lax.axis_index("c")
        s = lax.axis_index("s")
        wid = c * _NS + s
        base = wid * _SPT
        pltpu.sync_copy(h_hbm.at[pl.ds(base, _SPT)], strip_v)
        pltpu.sync_copy(b_hbm.at[pl.ds(base, _SPT)], gid_v)
        pltpu.sync_copy(negt_hbm, mx_v)
        pltpu.sync_copy(zerot_hbm, sm_v)
        pltpu.sync_copy(zerot_hbm, ct_v)
        lanes = lax.broadcasted_iota(jnp.int32, (16,), 0)
        lane0 = lanes == 0
        onev = jnp.ones((16,), jnp.float32)

        @pl.loop(0, _SPT, init_carry=jnp.zeros((16,), jnp.int32))
        def _(i, ivec):
            gidv = plsc.load_gather(gid_v, [ivec])
            plsc.addupdate_scatter(ct_v, [gidv, lanes], onev, mask=lane0)
            for kk in range(_H // 16):
                col = kk * 16 + lanes
                row = plsc.load_gather(strip_v, [ivec, col])
                cur = plsc.load_gather(mx_v, [gidv, col])
                plsc.store_scatter(mx_v, [gidv, col], jnp.maximum(cur, row))
                plsc.addupdate_scatter(sm_v, [gidv, col], row)
            return ivec + 1

        pltpu.sync_copy(mx_v, omax_hbm.at[wid])
        pltpu.sync_copy(sm_v, osum_hbm.at[wid])
        pltpu.sync_copy(ct_v, ocnt_hbm.at[wid])

    return k(h, batch_p1d, neg_table, zero_table)


def _tc_pool_reduce_mlp(maxs, sums, cnts, Wc1, bc1, Wc2, bc2, Wc3p, bc3p):
    def body(mx_ref, sm_ref, ct_ref, wc1_ref, bc1_ref, wc2_ref, bc2_ref,
             wc3_ref, bc3_ref, out_ref):
        mx = jnp.max(mx_ref[...], axis=0)[:_G]
        sm = jnp.sum(sm_ref[...], axis=0)[:_G]
        cnt = jnp.sum(ct_ref[...], axis=0)[:_G, 0:1]
        mean = sm / jnp.maximum(cnt, 1.0)
        pooled = jnp.concatenate([mean, mx], axis=1)
        z = jnp.dot(pooled, wc1_ref[...],
                    preferred_element_type=jnp.float32) + bc1_ref[...]
        z = jnp.maximum(z, 0.0)
        z = jnp.dot(z, wc2_ref[...],
                    preferred_element_type=jnp.float32) + bc2_ref[...]
        z = jnp.maximum(z, 0.0)
        out_ref[...] = jnp.dot(z, wc3_ref[...],
                               preferred_element_type=jnp.float32) + bc3_ref[...]

    return pl.pallas_call(
        body,
        out_shape=jax.ShapeDtypeStruct((_G, _H), jnp.float32),
    )(maxs, sums, cnts, Wc1, bc1, Wc2, bc2, Wc3p, bc3p)


def kernel(x, edge_index, batch, W_in, b_in, Wg, bg, Wc1, bc1, Wc2, bc2,
           Wc3, bc3):
    src = edge_index[0]
    dst = edge_index[1]
    npad_rows = _NPAD - _N
    pad_e = _EP - _E
    pad_src = (jnp.arange(pad_e, dtype=jnp.int32) % 1024)
    pad_dst = _N + (jnp.arange(pad_e, dtype=jnp.int32) % npad_rows)
    src_chunks = jnp.concatenate([src, pad_src]).reshape(_NW, _CH, _K)
    dst_chunks = jnp.concatenate([dst, pad_dst]).reshape(_NW, _CH, _K)

    ones_rows = jnp.ones((_K, _DW), jnp.float32)
    zeros_rows = jnp.zeros((_RPS, _H), jnp.float32)

    xp = jnp.pad(x, ((0, npad_rows), (0, 0)))
    batch_p1d = jnp.pad(batch, (0, npad_rows), constant_values=_G)
    neg_table = jnp.full((_GT, _H), float("-inf"), jnp.float32)
    zero_table = jnp.zeros((_GT, _H), jnp.float32)

    b_in2 = b_in.reshape(1, _H)
    bg2 = bg.reshape(_L, 1, _H)
    bc12 = bc1.reshape(1, _H)
    bc22 = bc2.reshape(1, _G)
    Wc3p = jnp.pad(Wc3, ((0, 0), (0, _H - _C)))
    bc3p = jnp.pad(bc3, (0, _H - _C)).reshape(1, _H)

    degp = _sc_degree(dst_chunks, ones_rows, zeros_rows)
    dinv, h, y = _tc_prologue(xp, W_in, b_in2, Wg[0], degp)
    for i in range(_L):
        p = _sc_msgpass(y, src_chunks, dst_chunks, zeros_rows)
        if i < _L - 1:
            h, y = _tc_combine(p, y, h, dinv, bg2[i], Wg[i + 1], True)
        else:
            (h,) = _tc_combine(p, y, h, dinv, bg2[i], Wg[i], False)

    maxs, sums, cnts = _sc_pool(h, batch_p1d, neg_table, zero_table)
    out = _tc_pool_reduce_mlp(maxs, sums, cnts, Wc1, bc12, Wc2, bc22, Wc3p,
                              bc3p)
    return out[:, :_C]
